# Initial kernel scaffold; baseline (speedup 1.0000x reference)
import jax
import jax.numpy as jnp
from jax.experimental import pallas as pl


def kernel(W, K, seq):
    raise NotImplementedError("write your pallas kernel here")


# speedup vs baseline: 5.5572x; 5.5572x over previous
import jax
import jax.numpy as jnp
from jax import lax
from jax.experimental import pallas as pl
from jax.experimental.pallas import tpu as pltpu
from jax.experimental.pallas import tpu_sc as plsc

_EMB = 64
_B = 1024
_L = 50
_NW = 32
_RPW = _B // _NW
_L3, _L5, _L7 = 48, 46, 44
_NCH = _EMB // 16


def _tec_body(W_hbm, K_hbm, seq_hbm, o3_hbm, o5_hbm, o7_hbm,
              seq_v, e_v, c_v, o3_v, o5_v, o7_v, sem_e, sem_c):
    cid = lax.axis_index("c")
    sid = lax.axis_index("s")
    wid = sid * 2 + cid
    base = wid * _RPW
    pltpu.sync_copy(seq_hbm.at[pl.ds(base, _RPW)], seq_v)

    def _row(g, carry):
        idx = seq_v.at[g]
        cp_e = pltpu.async_copy(W_hbm.at[idx], e_v, sem_e)
        cp_c = pltpu.async_copy(K_hbm.at[idx], c_v, sem_c)
        cp_e.wait()
        cp_c.wait()

        def _center(t, c2):
            for ch in range(_NCH):
                sl = pl.ds(ch * 16, 16)
                p = [e_v[t - 3 + j, sl] * c_v[t, j, sl] for j in range(7)]
                m3 = jnp.maximum(jnp.maximum(p[2], p[3]), p[4])
                m5 = jnp.maximum(jnp.maximum(m3, p[1]), p[5])
                m7 = jnp.maximum(jnp.maximum(m5, p[0]), p[6])
                o3_v[t - 1, sl] = m3
                o5_v[t - 2, sl] = m5
                o7_v[t - 3, sl] = m7
            return c2

        lax.fori_loop(3, 47, _center, 0)

        for t, have5 in ((1, False), (2, True), (47, True), (48, False)):
            js = range(1, 6) if have5 else range(2, 5)
            for ch in range(_NCH):
                sl = pl.ds(ch * 16, 16)
                p = {j: e_v[t - 3 + j, sl] * c_v[t, j, sl] for j in js}
                m3 = jnp.maximum(jnp.maximum(p[2], p[3]), p[4])
                o3_v[t - 1, sl] = m3
                if have5:
                    m5 = jnp.maximum(jnp.maximum(m3, p[1]), p[5])
                    o5_v[t - 2, sl] = m5

        b = base + g
        pltpu.sync_copy(o3_v, o3_hbm.at[b])
        pltpu.sync_copy(o5_v, o5_hbm.at[b])
        pltpu.sync_copy(o7_v, o7_hbm.at[b])
        return carry

    lax.fori_loop(0, _RPW, _row, 0)


def kernel(W, K, seq):
    mesh = plsc.VectorSubcoreMesh(core_axis_name="c", subcore_axis_name="s")
    out_type = (
        jax.ShapeDtypeStruct((_B, _L3, _EMB), jnp.float32),
        jax.ShapeDtypeStruct((_B, _L5, _EMB), jnp.float32),
        jax.ShapeDtypeStruct((_B, _L7, _EMB), jnp.float32),
    )
    scratch = [
        pltpu.VMEM((_RPW, _L), jnp.int32),
        pltpu.VMEM((_L, _EMB), jnp.float32),
        pltpu.VMEM((_L, 7, _EMB), jnp.float32),
        pltpu.VMEM((_L3, _EMB), jnp.float32),
        pltpu.VMEM((_L5, _EMB), jnp.float32),
        pltpu.VMEM((_L7, _EMB), jnp.float32),
        pltpu.SemaphoreType.DMA,
        pltpu.SemaphoreType.DMA,
    ]
    f = pl.kernel(_tec_body, mesh=mesh, out_type=out_type,
                  scratch_types=scratch,
                  compiler_params=pltpu.CompilerParams(
                      use_tc_tiling_on_sc=False))
    return f(W, K, seq)
